# Initial kernel scaffold; baseline (speedup 1.0000x reference)
import jax
import jax.numpy as jnp
from jax.experimental import pallas as pl


def kernel(x, edge_index, edge_attr, params):
    raise NotImplementedError("write your pallas kernel here")


# speedup vs baseline: 9.8540x; 9.8540x over previous
import functools

import jax
import jax.numpy as jnp
from jax import lax
from jax.scipy.linalg import block_diag
from jax.experimental import pallas as pl
from jax.experimental.pallas import tpu as pltpu
from jax.experimental.pallas import tpu_sc as plsc

_N = 10000
_E = 320000
_LAT = 32
_G = 4
_NP = 10240
_NC = 2
_NS = 16
_NW = _NC * _NS
_EPW = _E // _NW
_CH = 80
_SUP = 5
_SR = _CH * _SUP
_NSUP = _EPW // _SR
_NCH = _EPW // _CH
_RPT = _NP // _NS
_EQ = _E // _G
_NQ = _NP // _G
_BE = 4000

_F32 = jnp.float32


def _silu(x):
    return x * jax.nn.sigmoid(x)


def _dot(a, b):
    return jnp.dot(a, b, preferred_element_type=_F32)


def _enc_node_body(x_ref, w1, b1, w2, b2, ws, wd, h_ref, ts_ref, td_ref):
    t = _silu(_dot(x_ref[...], w1[...]) + b1[...])
    h = _dot(t, w2[...]) + b2[...]
    h_ref[...] = h
    ts_ref[...] = _dot(h, ws[...])
    td_ref[...] = _dot(h, wd[...])


def _edge0_body(ea_ref, gs_ref, gd_ref, ew1, eb1, ew2, eb2, w1, b1, w2, b2,
                o_ref):
    t = _silu(_dot(ea_ref[...], ew1[...]) + eb1[...])
    e0 = _dot(t, ew2[...]) + eb2[...]
    t2 = _silu(_dot(e0, w1[...]) + gs_ref[...] + gd_ref[...] + b1[...])
    o_ref[...] = e0 + _dot(t2, w2[...]) + b2[...]


def _edge_body(e_ref, gs_ref, gd_ref, w1, b1, w2, b2, o_ref):
    e = e_ref[...]
    t = _silu(_dot(e, w1[...]) + gs_ref[...] + gd_ref[...] + b1[...])
    o_ref[...] = e + _dot(t, w2[...]) + b2[...]


def _node_body(h_ref, a0_ref, a1_ref, wh, wa, b1, w2, b2, ws, wd,
               ho_ref, ts_ref, td_ref):
    h = h_ref[...]
    agg = a0_ref[...] + a1_ref[...]
    t = _silu(_dot(h, wh[...]) + _dot(agg, wa[...]) + b1[...])
    hn = h + _dot(t, w2[...]) + b2[...]
    ho_ref[...] = hn
    ts_ref[...] = _dot(hn, ws[...])
    td_ref[...] = _dot(hn, wd[...])


def _node_dec_body(h_ref, a0_ref, a1_ref, wh, wa, b1, w2, b2,
                   dw1, db1, dw2, db2, u_ref):
    h = h_ref[...]
    agg = a0_ref[...] + a1_ref[...]
    t = _silu(_dot(h, wh[...]) + _dot(agg, wa[...]) + b1[...])
    hn = h + _dot(t, w2[...]) + b2[...]
    t2 = _silu(_dot(hn, dw1[...]) + db1[...])
    u_ref[...] = _dot(t2, dw2[...]) + db2[...]


def _full_call(body, out_shapes):
    return pl.pallas_call(
        body,
        out_shape=[jax.ShapeDtypeStruct(s, _F32) for s in out_shapes],
    )


def _sc_mesh():
    return plsc.VectorSubcoreMesh(
        core_axis_name="c", subcore_axis_name="s",
        num_cores=_NC, num_subcores=_NS)


def _gather_body(ts_hbm, td_hbm, src_hbm, dst_hbm, gs_hbm, gd_hbm,
                 idx_s, idx_d, bs0, bd0, bs1, bd1, sem0, sem1):
    c = lax.axis_index("c")
    s = lax.axis_index("s")
    wid = s * _NC + c
    base = wid * _EPW
    pltpu.sync_copy(src_hbm.at[wid], idx_s)
    pltpu.sync_copy(dst_hbm.at[wid], idx_d)

    def body(sup, carry):
        cps = []
        for k in range(_SUP):
            j = sup * _SUP + k
            cps.append(pltpu.async_copy(ts_hbm.at[idx_s.at[j]],
                                        bs0.at[pl.ds(k * _CH, _CH)], sem0))
            cps.append(pltpu.async_copy(td_hbm.at[idx_d.at[j]],
                                        bd0.at[pl.ds(k * _CH, _CH)], sem0))
        for cp in cps:
            cp.wait()
        pltpu.sync_copy(bs0, gs_hbm.at[pl.ds(base + sup * _SR, _SR)])
        pltpu.sync_copy(bd0, gd_hbm.at[pl.ds(base + sup * _SR, _SR)])
        return carry

    lax.fori_loop(0, _NSUP, body, 0)


def _scatter_body(e_hbm, dst_hbm, a0_hbm, a1_hbm,
                  idx_d, eb0, eb1, ob, agg_sh, seml0, seml1, sems0, sems1):
    c = lax.axis_index("c")
    s = lax.axis_index("s")
    wid = s * _NC + c
    base = wid * _EPW
    pltpu.sync_copy(dst_hbm.at[wid], idx_d)

    zeros = jnp.zeros((16,), _F32)

    def zb(i, carry):
        ob[i, 0:16] = zeros
        ob[i, 16:32] = zeros
        return carry

    lax.fori_loop(0, _RPT, zb, 0)
    pltpu.sync_copy(ob, agg_sh.at[pl.ds(s * _RPT, _RPT)])
    plsc.subcore_barrier()

    def body(sup, carry):
        pltpu.async_copy(e_hbm.at[pl.ds(base + sup * _SR, _SR)],
                         eb0, seml0).wait()
        cps = []
        for k in range(_SUP):
            j = sup * _SUP + k
            cps.append(pltpu.async_copy(eb0.at[pl.ds(k * _CH, _CH)],
                                        agg_sh.at[idx_d.at[j]], sems0,
                                        add=True))
        for cp in cps:
            cp.wait()
        return carry

    lax.fori_loop(0, _NSUP, body, 0)
    plsc.subcore_barrier()

    pltpu.sync_copy(agg_sh.at[pl.ds(s * _RPT, _RPT)], ob)

    @pl.when(c == 0)
    def _():
        pltpu.sync_copy(ob, a0_hbm.at[pl.ds(s * _RPT, _RPT)])

    @pl.when(c == 1)
    def _():
        pltpu.sync_copy(ob, a1_hbm.at[pl.ds(s * _RPT, _RPT)])


def _make_sc_calls():
    mesh = _sc_mesh()
    params = pltpu.CompilerParams(use_tc_tiling_on_sc=False)
    gather = functools.partial(
        pl.kernel,
        _gather_body,
        out_type=[jax.ShapeDtypeStruct((_E, _LAT), _F32),
                  jax.ShapeDtypeStruct((_E, _LAT), _F32)],
        mesh=mesh,
        compiler_params=params,
        scratch_types=[
            pltpu.VMEM((_NCH, _CH), jnp.int32),
            pltpu.VMEM((_NCH, _CH), jnp.int32),
            pltpu.VMEM((_SR, _LAT), _F32),
            pltpu.VMEM((_SR, _LAT), _F32),
            pltpu.VMEM((_SR, _LAT), _F32),
            pltpu.VMEM((_SR, _LAT), _F32),
            pltpu.SemaphoreType.DMA,
            pltpu.SemaphoreType.DMA,
        ],
    )()
    scatter = functools.partial(
        pl.kernel,
        _scatter_body,
        out_type=[jax.ShapeDtypeStruct((_NP, _LAT), _F32),
                  jax.ShapeDtypeStruct((_NP, _LAT), _F32)],
        mesh=mesh,
        compiler_params=params,
        scratch_types=[
            pltpu.VMEM((_NCH, _CH), jnp.int32),
            pltpu.VMEM((_SR, _LAT), _F32),
            pltpu.VMEM((_SR, _LAT), _F32),
            pltpu.VMEM((_RPT, _LAT), _F32),
            pltpu.VMEM_SHARED((_NP, _LAT), _F32),
            pltpu.SemaphoreType.DMA,
            pltpu.SemaphoreType.DMA,
            pltpu.SemaphoreType.DMA,
            pltpu.SemaphoreType.DMA,
        ],
    )()
    return gather, scatter


def kernel(x, edge_index, edge_attr, params):
    bd = lambda w: block_diag(w, w, w, w)
    tl = lambda b: jnp.tile(b, _G)[None, :]

    enc_n, enc_e, dec = params['enc_node'], params['enc_edge'], params['dec_node']
    en_w1, en_b1 = bd(enc_n[0]['W']), tl(enc_n[0]['b'])
    en_w2, en_b2 = bd(enc_n[1]['W']), tl(enc_n[1]['b'])
    ee_w1, ee_b1 = bd(enc_e[0]['W']), tl(enc_e[0]['b'])
    ee_w2, ee_b2 = bd(enc_e[1]['W']), tl(enc_e[1]['b'])
    d_w1, d_b1 = bd(dec[0]['W']), tl(dec[0]['b'])
    d_w2, d_b2 = bd(dec[1]['W']), tl(dec[1]['b'])
    blks = []
    for blk in params['proc']:
        w1 = blk['edge'][0]['W']
        wn = blk['node'][0]['W']
        blks.append(dict(
            w1e=bd(w1[:_LAT]), w1s=bd(w1[_LAT:2 * _LAT]), w1d=bd(w1[2 * _LAT:]),
            b1=tl(blk['edge'][0]['b']),
            w2=bd(blk['edge'][1]['W']), b2=tl(blk['edge'][1]['b']),
            wnh=bd(wn[:_LAT]), wna=bd(wn[_LAT:]),
            bn1=tl(blk['node'][0]['b']),
            wn2=bd(blk['node'][1]['W']), bn2=tl(blk['node'][1]['b']),
        ))

    xp = jnp.pad(x, ((0, _NP - _N), (0, 0))).reshape(_NQ, _G * 128)
    ea4 = edge_attr.reshape(_EQ, _G * 16)
    src3 = edge_index[0].reshape(_NW, _NCH, _CH)
    dst3 = edge_index[1].reshape(_NW, _NCH, _CH)

    gather, scatter = _make_sc_calls()

    enc_call = _full_call(_enc_node_body,
                          [(_NQ, 128), (_NQ, 128), (_NQ, 128)])
    h, ts, td = enc_call(xp, en_w1, en_b1, en_w2, en_b2,
                         blks[0]['w1s'], blks[0]['w1d'])

    e = None
    for p in range(8):
        blk = blks[p]
        gs, gd = gather(ts.reshape(_NP, _LAT), td.reshape(_NP, _LAT),
                        src3, dst3)
        gs4 = gs.reshape(_EQ, 128)
        gd4 = gd.reshape(_EQ, 128)
        if p == 0:
            call = pl.pallas_call(
                _edge0_body,
                grid=(_EQ // _BE,),
                in_specs=[
                    pl.BlockSpec((_BE, 64), lambda i: (i, 0)),
                    pl.BlockSpec((_BE, 128), lambda i: (i, 0)),
                    pl.BlockSpec((_BE, 128), lambda i: (i, 0)),
                    pl.BlockSpec((64, 128), lambda i: (0, 0)),
                    pl.BlockSpec((1, 128), lambda i: (0, 0)),
                    pl.BlockSpec((128, 128), lambda i: (0, 0)),
                    pl.BlockSpec((1, 128), lambda i: (0, 0)),
                    pl.BlockSpec((128, 128), lambda i: (0, 0)),
                    pl.BlockSpec((1, 128), lambda i: (0, 0)),
                    pl.BlockSpec((128, 128), lambda i: (0, 0)),
                    pl.BlockSpec((1, 128), lambda i: (0, 0)),
                ],
                out_specs=pl.BlockSpec((_BE, 128), lambda i: (i, 0)),
                out_shape=jax.ShapeDtypeStruct((_EQ, 128), _F32),
            )
            e = call(ea4, gs4, gd4, ee_w1, ee_b1, ee_w2, ee_b2,
                     blk['w1e'], blk['b1'], blk['w2'], blk['b2'])
        else:
            call = pl.pallas_call(
                _edge_body,
                grid=(_EQ // _BE,),
                in_specs=[
                    pl.BlockSpec((_BE, 128), lambda i: (i, 0)),
                    pl.BlockSpec((_BE, 128), lambda i: (i, 0)),
                    pl.BlockSpec((_BE, 128), lambda i: (i, 0)),
                    pl.BlockSpec((128, 128), lambda i: (0, 0)),
                    pl.BlockSpec((1, 128), lambda i: (0, 0)),
                    pl.BlockSpec((128, 128), lambda i: (0, 0)),
                    pl.BlockSpec((1, 128), lambda i: (0, 0)),
                ],
                out_specs=pl.BlockSpec((_BE, 128), lambda i: (i, 0)),
                out_shape=jax.ShapeDtypeStruct((_EQ, 128), _F32),
            )
            e = call(e, gs4, gd4, blk['w1e'], blk['b1'], blk['w2'], blk['b2'])

        a0, a1 = scatter(e.reshape(_E, _LAT), dst3)
        a04 = a0.reshape(_NQ, 128)
        a14 = a1.reshape(_NQ, 128)
        if p < 7:
            nblk = blks[p + 1]
            node_call = _full_call(_node_body,
                                   [(_NQ, 128), (_NQ, 128), (_NQ, 128)])
            h, ts, td = node_call(h, a04, a14, blk['wnh'], blk['wna'],
                                  blk['bn1'], blk['wn2'], blk['bn2'],
                                  nblk['w1s'], nblk['w1d'])
        else:
            dec_call = _full_call(_node_dec_body, [(_NQ, _G)])
            (u4,) = dec_call(h, a04, a14, blk['wnh'], blk['wna'],
                             blk['bn1'], blk['wn2'], blk['bn2'],
                             d_w1, d_b1, d_w2, d_b2)

    return u4.reshape(_NP, 1)[:_N]
